# baseline (speedup 1.0000x reference)
import jax
import jax.numpy as jnp
from jax import lax
from jax.experimental import pallas as pl
from jax.experimental.pallas import tpu as pltpu
from jax.experimental.pallas import tpu_sc as plsc

_B, _N, _K, _NUM_PAIR = 4, 256, 128, 512
_M = _B * _N * _N
_STD_WIDTH = 1.0

_NC, _NS, _L = 2, 16, 16
_NW = _NC * _NS
_G = 128
_NBLK = _M // (_K * _G)
_ABLK = 2
_SA = _ABLK * _K * _G
_SB = _M - _SA
_CH_A = _SA // _NW
_CH_B = _SB // _NW


def _make_sc_body(start, chunk):
    def body(x_hbm, idx_hbm, mul_hbm, bias_hbm, out_hbm,
             mul_v, bias_v, idx_v, x_v, xt_v):
        wid = lax.axis_index("s") * _NC + lax.axis_index("c")
        base = start + wid * chunk
        pltpu.sync_copy(mul_hbm, mul_v)
        pltpu.sync_copy(bias_hbm, bias_v)
        pltpu.sync_copy(idx_hbm.at[pl.ds(base, chunk)], idx_v)
        pltpu.sync_copy(x_hbm.at[pl.ds(base, chunk)], x_v)

        def step(i, carry):
            sl = pl.ds(i * _L, _L)
            idx = idx_v[sl]
            xv = x_v[sl]
            mv = plsc.load_gather(mul_v, [idx])
            bv = plsc.load_gather(bias_v, [idx])
            xt_v[sl] = jnp.abs(mv) * xv + bv
            return carry

        lax.fori_loop(0, chunk // _L, step, 0)
        pltpu.sync_copy(xt_v, out_hbm.at[pl.ds(wid * chunk, chunk)])

    return body


def _sc_gather(xf, idx, mul_f, bias_f, start, size):
    chunk = size // _NW
    mesh = plsc.VectorSubcoreMesh(core_axis_name="c", subcore_axis_name="s")
    fn = pl.kernel(
        _make_sc_body(start, chunk),
        mesh=mesh,
        out_type=jax.ShapeDtypeStruct((size,), jnp.float32),
        scratch_types=[
            pltpu.VMEM((_NUM_PAIR,), jnp.float32),
            pltpu.VMEM((_NUM_PAIR,), jnp.float32),
            pltpu.VMEM((chunk,), jnp.int32),
            pltpu.VMEM((chunk,), jnp.float32),
            pltpu.VMEM((chunk,), jnp.float32),
        ],
        compiler_params=pltpu.CompilerParams(needs_layout_passes=False),
    )
    return fn(xf, idx, mul_f, bias_f)


def _expand(mean_ref, xt_ref, out_ref):
    log2e = 1.4426950408889634
    std = (mean_ref[0, 0, 1] - mean_ref[0, 0, 0]) * _STD_WIDTH
    neg2 = (-0.5 / (std * std)) * log2e
    c2 = -jnp.log(((2.0 * 3.14159) ** 0.5) * std) * log2e
    col = xt_ref[:, :][:, :, None]
    d = col - mean_ref[:, :, :]
    out_ref[:, :, :] = jnp.exp2((neg2 * d) * d + c2)


def _tc_body_a(mean_ref, xt_ref, out_ref):
    _expand(mean_ref, xt_ref, out_ref)


def _tc_body_b(mean_ref, xt_ref, buf_ref, out_ref):
    del buf_ref
    _expand(mean_ref, xt_ref, out_ref)


def _tc_expand_a(xt_a, mean):
    return pl.pallas_call(
        _tc_body_a,
        grid=(_ABLK,),
        in_specs=[
            pl.BlockSpec((1, 1, _K), lambda i: (0, 0, 0)),
            pl.BlockSpec((_G, _K), lambda i: (i, 0)),
        ],
        out_specs=pl.BlockSpec((_G, _K, _K), lambda i: (i, 0, 0)),
        out_shape=jax.ShapeDtypeStruct((_M // _K, _K, _K), jnp.float32),
    )(mean.reshape(1, 1, _K), xt_a.reshape(_SA // _K, _K))


def _tc_expand_b(xt_b, mean, buf):
    return pl.pallas_call(
        _tc_body_b,
        grid=(_NBLK - _ABLK,),
        in_specs=[
            pl.BlockSpec((1, 1, _K), lambda i: (0, 0, 0)),
            pl.BlockSpec((_G, _K), lambda i: (i, 0)),
            pl.BlockSpec(memory_space=pl.ANY),
        ],
        out_specs=pl.BlockSpec((_G, _K, _K), lambda i: (i + _ABLK, 0, 0)),
        out_shape=jax.ShapeDtypeStruct((_M // _K, _K, _K), jnp.float32),
        input_output_aliases={2: 0},
    )(mean.reshape(1, 1, _K), xt_b.reshape(_SB // _K, _K), buf)


def kernel(x, atom_pair, mul_weight, bias_weight, mean):
    xf = x.reshape(_M)
    idx = atom_pair.reshape(_M).astype(jnp.int32)
    mul_f = mul_weight.reshape(_NUM_PAIR)
    bias_f = bias_weight.reshape(_NUM_PAIR)
    xt_a = _sc_gather(xf, idx, mul_f, bias_f, 0, _SA)
    xt_b = _sc_gather(xf, idx, mul_f, bias_f, _SA, _SB)
    buf = _tc_expand_a(xt_a, mean)
    out = _tc_expand_b(xt_b, mean, buf)
    return out.reshape(_B, _N, _N, _K)


# speedup vs baseline: 1.0256x; 1.0256x over previous
import functools

import jax
import jax.numpy as jnp
from jax import lax
from jax.experimental import pallas as pl
from jax.experimental.pallas import tpu as pltpu
from jax.experimental.pallas import tpu_sc as plsc

_B, _N, _K, _NUM_PAIR = 4, 256, 128, 512
_M = _B * _N * _N
_STD_WIDTH = 1.0

_NC, _NS, _L = 2, 16, 16
_NW = _NC * _NS
_CHUNK = _M // _NW
_G = 64


def _sc_gather_body(x_hbm, idx_hbm, mul_hbm, bias_hbm, out_hbm,
                    mul_v, bias_v, idx_v, x_v, xt_v):
    wid = lax.axis_index("s") * _NC + lax.axis_index("c")
    base = wid * _CHUNK
    pltpu.sync_copy(mul_hbm, mul_v)
    pltpu.sync_copy(bias_hbm, bias_v)
    pltpu.sync_copy(idx_hbm.at[pl.ds(base, _CHUNK)], idx_v)
    pltpu.sync_copy(x_hbm.at[pl.ds(base, _CHUNK)], x_v)

    def body(i, carry):
        sl = pl.ds(i * _L, _L)
        idx = idx_v[sl]
        xv = x_v[sl]
        mv = plsc.load_gather(mul_v, [idx])
        bv = plsc.load_gather(bias_v, [idx])
        xt_v[sl] = jnp.abs(mv) * xv + bv
        return carry

    lax.fori_loop(0, _CHUNK // _L, body, 0)
    pltpu.sync_copy(xt_v, out_hbm.at[pl.ds(base, _CHUNK)])


def _sc_gather(xf, idx, mul_f, bias_f):
    mesh = plsc.VectorSubcoreMesh(core_axis_name="c", subcore_axis_name="s")
    fn = pl.kernel(
        _sc_gather_body,
        mesh=mesh,
        out_type=jax.ShapeDtypeStruct((_M,), jnp.float32),
        scratch_types=[
            pltpu.VMEM((_NUM_PAIR,), jnp.float32),
            pltpu.VMEM((_NUM_PAIR,), jnp.float32),
            pltpu.VMEM((_CHUNK,), jnp.int32),
            pltpu.VMEM((_CHUNK,), jnp.float32),
            pltpu.VMEM((_CHUNK,), jnp.float32),
        ],
        compiler_params=pltpu.CompilerParams(needs_layout_passes=False),
    )
    return fn(xf, idx, mul_f, bias_f)


def _tc_expand_body(mean_ref, xt_ref, out_ref):
    log2e = 1.4426950408889634
    std = (mean_ref[0, 0, 1] - mean_ref[0, 0, 0]) * _STD_WIDTH
    neg2 = (-0.5 / (std * std)) * log2e
    c2 = -jnp.log(((2.0 * 3.14159) ** 0.5) * std) * log2e
    col = xt_ref[:, :][:, :, None]
    d = col - mean_ref[:, :, :]
    out_ref[:, :, :] = jnp.exp2((neg2 * d) * d + c2)


def _tc_expand(xt_flat, mean, interpret=False):
    return pl.pallas_call(
        _tc_expand_body,
        grid=(_M // (_G * _K),),
        in_specs=[
            pl.BlockSpec((1, 1, _K), lambda i: (0, 0, 0)),
            pl.BlockSpec((_G, _K), lambda i: (i, 0)),
        ],
        out_specs=pl.BlockSpec((_G, _K, _K), lambda i: (i, 0, 0)),
        out_shape=jax.ShapeDtypeStruct((_M // _K, _K, _K), jnp.float32),
        interpret=interpret,
    )(mean.reshape(1, 1, _K), xt_flat.reshape(_M // _K, _K))


def kernel(x, atom_pair, mul_weight, bias_weight, mean):
    xf = x.reshape(_M)
    idx = atom_pair.reshape(_M).astype(jnp.int32)
    mul_f = mul_weight.reshape(_NUM_PAIR)
    bias_f = bias_weight.reshape(_NUM_PAIR)
    xt = _sc_gather(xf, idx, mul_f, bias_f)
    out = _tc_expand(xt, mean)
    return out.reshape(_B, _N, _N, _K)


# speedup vs baseline: 1.1278x; 1.0996x over previous
import functools

import jax
import jax.numpy as jnp
from jax import lax
from jax.experimental import pallas as pl
from jax.experimental.pallas import tpu as pltpu
from jax.experimental.pallas import tpu_sc as plsc

_B, _N, _K, _NUM_PAIR = 4, 256, 128, 512
_M = _B * _N * _N
_STD_WIDTH = 1.0

_NC, _NS, _L = 2, 16, 16
_NW = _NC * _NS
_CHUNK = _M // _NW
_G = 128


def _sc_gather_body(x_hbm, idx_hbm, mul_hbm, bias_hbm, out_hbm,
                    mul_v, bias_v, idx_v, x_v, xt_v):
    wid = lax.axis_index("s") * _NC + lax.axis_index("c")
    base = wid * _CHUNK
    pltpu.sync_copy(mul_hbm, mul_v)
    pltpu.sync_copy(bias_hbm, bias_v)
    pltpu.sync_copy(idx_hbm.at[pl.ds(base, _CHUNK)], idx_v)
    pltpu.sync_copy(x_hbm.at[pl.ds(base, _CHUNK)], x_v)

    @plsc.parallel_loop(0, _CHUNK // _L, unroll=4)
    def body(i):
        sl = pl.ds(i * _L, _L)
        idx = idx_v[sl]
        xv = x_v[sl]
        mv = plsc.load_gather(mul_v, [idx])
        bv = plsc.load_gather(bias_v, [idx])
        xt_v[sl] = jnp.abs(mv) * xv + bv

    pltpu.sync_copy(xt_v, out_hbm.at[pl.ds(base, _CHUNK)])


def _sc_gather(xf, idx, mul_f, bias_f):
    mesh = plsc.VectorSubcoreMesh(core_axis_name="c", subcore_axis_name="s")
    fn = pl.kernel(
        _sc_gather_body,
        mesh=mesh,
        out_type=jax.ShapeDtypeStruct((_M,), jnp.float32),
        scratch_types=[
            pltpu.VMEM((_NUM_PAIR,), jnp.float32),
            pltpu.VMEM((_NUM_PAIR,), jnp.float32),
            pltpu.VMEM((_CHUNK,), jnp.int32),
            pltpu.VMEM((_CHUNK,), jnp.float32),
            pltpu.VMEM((_CHUNK,), jnp.float32),
        ],
        compiler_params=pltpu.CompilerParams(needs_layout_passes=False),
    )
    return fn(xf, idx, mul_f, bias_f)


def _tc_expand_body(mean_ref, xt_ref, out_ref):
    log2e = 1.4426950408889634
    std = (mean_ref[0, 0, 1] - mean_ref[0, 0, 0]) * _STD_WIDTH
    neg2 = (-0.5 / (std * std)) * log2e
    c2 = -jnp.log(((2.0 * 3.14159) ** 0.5) * std) * log2e
    col = xt_ref[:, :][:, :, None]
    d = col - mean_ref[:, :, :]
    out_ref[:, :, :] = jnp.exp2((neg2 * d) * d + c2)


def _tc_expand(xt_flat, mean, interpret=False):
    return pl.pallas_call(
        _tc_expand_body,
        grid=(_M // (_G * _K),),
        in_specs=[
            pl.BlockSpec((1, 1, _K), lambda i: (0, 0, 0)),
            pl.BlockSpec((_G, _K), lambda i: (i, 0)),
        ],
        out_specs=pl.BlockSpec((_G, _K, _K), lambda i: (i, 0, 0)),
        out_shape=jax.ShapeDtypeStruct((_M // _K, _K, _K), jnp.float32),
        interpret=interpret,
    )(mean.reshape(1, 1, _K), xt_flat.reshape(_M // _K, _K))


def kernel(x, atom_pair, mul_weight, bias_weight, mean):
    xf = x.reshape(_M)
    idx = atom_pair.reshape(_M).astype(jnp.int32)
    mul_f = mul_weight.reshape(_NUM_PAIR)
    bias_f = bias_weight.reshape(_NUM_PAIR)
    xt = _sc_gather(xf, idx, mul_f, bias_f)
    out = _tc_expand(xt, mean)
    return out.reshape(_B, _N, _N, _K)
